# baseline (speedup 1.0000x reference)
import functools

import jax
import jax.numpy as jnp
from jax import lax
from jax.experimental import pallas as pl
from jax.experimental.pallas import tpu as pltpu
from jax.experimental.pallas import tpu_sc as plsc

_EPS = 1e-3
_NV = 15000
_NF = 30000
_NQ = 50000

_NTILES = 32
_FPAD = 30720
_FPT = _FPAD // _NTILES
_NSTEP = _FPT // 16

_NQB = 256
_NQPAD = 50176
_NQBLKS = _NQPAD // _NQB
_FB = 2048
_NFB = _FPAD // _FB


def _face_prep_body(pos_hbm, f0_hbm, f1_hbm, f2_hbm,
                    o0, o1, o2, o3, o4, o5, o6, o7,
                    pos_v, f0_v, f1_v, f2_v,
                    r0, r1, r2, r3, r4, r5, r6, r7):
    wid = lax.axis_index("s") * 2 + lax.axis_index("c")
    base = wid * _FPT
    pltpu.sync_copy(pos_hbm, pos_v)
    pltpu.sync_copy(f0_hbm.at[pl.ds(base, _FPT)], f0_v)
    pltpu.sync_copy(f1_hbm.at[pl.ds(base, _FPT)], f1_v)
    pltpu.sync_copy(f2_hbm.at[pl.ds(base, _FPT)], f2_v)

    def step(i, carry):
        s = i * 16
        v0 = f0_v[pl.ds(s, 16)] * 3
        v1 = f1_v[pl.ds(s, 16)] * 3
        v2 = f2_v[pl.ds(s, 16)] * 3
        x0 = plsc.load_gather(pos_v, [v0])
        y0 = plsc.load_gather(pos_v, [v0 + 1])
        z0 = plsc.load_gather(pos_v, [v0 + 2])
        x1 = plsc.load_gather(pos_v, [v1])
        y1 = plsc.load_gather(pos_v, [v1 + 1])
        z1 = plsc.load_gather(pos_v, [v1 + 2])
        x2 = plsc.load_gather(pos_v, [v2])
        y2 = plsc.load_gather(pos_v, [v2 + 1])
        z2 = plsc.load_gather(pos_v, [v2 + 2])
        third = jnp.float32(1.0 / 3.0)
        cx = (x0 + x1 + x2) * third
        cy = (y0 + y1 + y2) * third
        cz = (z0 + z1 + z2) * third
        e1x = x1 - x0
        e1y = y1 - y0
        e1z = z1 - z0
        e2x = x2 - x0
        e2y = y2 - y0
        e2z = z2 - z0
        nx = e1y * e2z - e1z * e2y
        ny = e1z * e2x - e1x * e2z
        nz = e1x * e2y - e1y * e2x
        ss = nx * nx + ny * ny + nz * nz
        ib = plsc.bitcast(ss, jnp.int32)
        ib = jnp.int32(0x5F3759DF) - (ib >> 1)
        yx = plsc.bitcast(ib, jnp.float32)
        h = ss * jnp.float32(0.5)
        yx = yx * (jnp.float32(1.5) - h * yx * yx)
        yx = yx * (jnp.float32(1.5) - h * yx * yx)
        yx = yx * (jnp.float32(1.5) - h * yx * yx)
        norm = ss * yx
        scale = jnp.float32(1.0) / (norm + jnp.float32(1e-12))
        nx = nx * scale
        ny = ny * scale
        nz = nz * scale
        nb = -(cx * nx + cy * ny + cz * nz)
        pn = cx * cx + cy * cy + cz * cz
        gidx = base + s + lax.iota(jnp.int32, 16)
        pn = jnp.where(gidx < _NF, pn, jnp.float32(1e30))
        m2 = jnp.float32(-2.0)
        r0[pl.ds(s, 16)] = cx * m2
        r1[pl.ds(s, 16)] = cy * m2
        r2[pl.ds(s, 16)] = cz * m2
        r3[pl.ds(s, 16)] = pn
        r4[pl.ds(s, 16)] = nx
        r5[pl.ds(s, 16)] = ny
        r6[pl.ds(s, 16)] = nz
        r7[pl.ds(s, 16)] = nb
        return carry

    lax.fori_loop(0, _NSTEP, step, 0)
    pltpu.sync_copy(r0, o0.at[pl.ds(base, _FPT)])
    pltpu.sync_copy(r1, o1.at[pl.ds(base, _FPT)])
    pltpu.sync_copy(r2, o2.at[pl.ds(base, _FPT)])
    pltpu.sync_copy(r3, o3.at[pl.ds(base, _FPT)])
    pltpu.sync_copy(r4, o4.at[pl.ds(base, _FPT)])
    pltpu.sync_copy(r5, o5.at[pl.ds(base, _FPT)])
    pltpu.sync_copy(r6, o6.at[pl.ds(base, _FPT)])
    pltpu.sync_copy(r7, o7.at[pl.ds(base, _FPT)])


_face_prep = pl.kernel(
    _face_prep_body,
    [jax.ShapeDtypeStruct((_FPAD,), jnp.float32)] * 8,
    mesh=plsc.VectorSubcoreMesh(core_axis_name="c", subcore_axis_name="s"),
    compiler_params=pltpu.CompilerParams(needs_layout_passes=False),
    scratch_types=(
        [pltpu.VMEM((_NV * 3,), jnp.float32)]
        + [pltpu.VMEM((_FPT,), jnp.int32)] * 3
        + [pltpu.VMEM((_FPT,), jnp.float32)] * 8
    ),
)


def _knn_body(q_ref, c0, c1, c2, c3, c4, c5, c6, c7,
              loss_ref, perc_ref, bs_ref, bd_ref):
    i = pl.program_id(0)
    j = pl.program_id(1)
    q = q_ref[...]
    qx = q[:, 0:1]
    qy = q[:, 1:2]
    qz = q[:, 2:3]
    s_blk = qx * c0[...] + qy * c1[...] + qz * c2[...] + c3[...]
    d_blk = qx * c4[...] + qy * c5[...] + qz * c6[...] + c7[...]
    m = jnp.min(s_blk, axis=1, keepdims=True)
    dm = jnp.max(jnp.where(s_blk == m, d_blk, -jnp.inf), axis=1, keepdims=True)

    @pl.when(j == 0)
    def _():
        bs_ref[...] = m
        bd_ref[...] = dm

    @pl.when(j > 0)
    def _():
        upd = m < bs_ref[...]
        bs_ref[...] = jnp.where(upd, m, bs_ref[...])
        bd_ref[...] = jnp.where(upd, dm, bd_ref[...])

    @pl.when(j == _NFB - 1)
    def _():
        d = bd_ref[...]
        gq = i * _NQB + lax.broadcasted_iota(jnp.int32, (_NQB, 1), 0)
        interp = jnp.maximum(jnp.float32(_EPS) - d, 0.0)
        interp = jnp.where(gq < _NQ, interp, 0.0)
        part_l = jnp.sum(interp * interp * interp)
        part_c = jnp.sum((interp > 0).astype(jnp.float32))
        prev_l = jnp.where(i == 0, 0.0, loss_ref[0, 0])
        prev_c = jnp.where(i == 0, 0.0, perc_ref[0, 0])
        loss_ref[0, 0] = prev_l + part_l
        perc_ref[0, 0] = prev_c + part_c

    @pl.when((j == _NFB - 1) & (i == _NQBLKS - 1))
    def _():
        perc_ref[0, 0] = perc_ref[0, 0] * jnp.float32(1.0 / _NQ)


_knn_call = pl.pallas_call(
    _knn_body,
    grid=(_NQBLKS, _NFB),
    in_specs=(
        [pl.BlockSpec((_NQB, 3), lambda i, j: (i, 0))]
        + [pl.BlockSpec((1, _FB), lambda i, j: (0, j))] * 8
    ),
    out_specs=[pl.BlockSpec(memory_space=pltpu.SMEM),
               pl.BlockSpec(memory_space=pltpu.SMEM)],
    out_shape=[jax.ShapeDtypeStruct((1, 1), jnp.float32),
               jax.ShapeDtypeStruct((1, 1), jnp.float32)],
    scratch_shapes=[pltpu.VMEM((_NQB, 1), jnp.float32),
                    pltpu.VMEM((_NQB, 1), jnp.float32)],
)


def kernel(obstacle_pos, pred_pos, obstacle_faces):
    pos_flat = obstacle_pos.reshape(-1)
    fpad = jnp.pad(obstacle_faces, ((0, _FPAD - _NF), (0, 0)))
    f0 = fpad[:, 0]
    f1 = fpad[:, 1]
    f2 = fpad[:, 2]
    cols = _face_prep(pos_flat, f0, f1, f2)
    cols2d = [c.reshape(1, _FPAD) for c in cols]
    qpad = jnp.pad(pred_pos, ((0, _NQPAD - _NQ), (0, 0)))
    loss2, perc2 = _knn_call(qpad, *cols2d)
    return loss2[0, 0], perc2[0, 0]


# speedup vs baseline: 1.3964x; 1.3964x over previous
import jax
import jax.numpy as jnp
from jax import lax
from jax.experimental import pallas as pl
from jax.experimental.pallas import tpu as pltpu
from jax.experimental.pallas import tpu_sc as plsc

_EPS = 1e-3
_NV = 15000
_NF = 30000
_NQ = 50000

_NTILES = 32
_FPAD = 30720
_FPT = _FPAD // _NTILES
_NSTEP = _FPT // 16
_TW = 16

_NQB = 256
_NQPAD = 50176
_NQBLKS = _NQPAD // _NQB
_FB = 2048
_NFB = _FPAD // _FB

_QPT = _NQPAD // _NTILES
_GW = 112
_GROWS = _QPT // _GW
_QSTEP = _QPT // 16


def _face_prep_body(pos_hbm, f0_hbm, f1_hbm, f2_hbm,
                    o0, o1, o2, o3, otbl,
                    pos_v, f0_v, f1_v, f2_v,
                    r0, r1, r2, r3, tbl_v):
    wid = lax.axis_index("s") * 2 + lax.axis_index("c")
    base = wid * _FPT
    pltpu.sync_copy(pos_hbm, pos_v)
    pltpu.sync_copy(f0_hbm.at[pl.ds(base, _FPT)], f0_v)
    pltpu.sync_copy(f1_hbm.at[pl.ds(base, _FPT)], f1_v)
    pltpu.sync_copy(f2_hbm.at[pl.ds(base, _FPT)], f2_v)

    def step(i, carry):
        s = i * 16
        v0 = f0_v[pl.ds(s, 16)] * 3
        v1 = f1_v[pl.ds(s, 16)] * 3
        v2 = f2_v[pl.ds(s, 16)] * 3
        x0 = plsc.load_gather(pos_v, [v0])
        y0 = plsc.load_gather(pos_v, [v0 + 1])
        z0 = plsc.load_gather(pos_v, [v0 + 2])
        x1 = plsc.load_gather(pos_v, [v1])
        y1 = plsc.load_gather(pos_v, [v1 + 1])
        z1 = plsc.load_gather(pos_v, [v1 + 2])
        x2 = plsc.load_gather(pos_v, [v2])
        y2 = plsc.load_gather(pos_v, [v2 + 1])
        z2 = plsc.load_gather(pos_v, [v2 + 2])
        third = jnp.float32(1.0 / 3.0)
        cx = (x0 + x1 + x2) * third
        cy = (y0 + y1 + y2) * third
        cz = (z0 + z1 + z2) * third
        e1x = x1 - x0
        e1y = y1 - y0
        e1z = z1 - z0
        e2x = x2 - x0
        e2y = y2 - y0
        e2z = z2 - z0
        nx = e1y * e2z - e1z * e2y
        ny = e1z * e2x - e1x * e2z
        nz = e1x * e2y - e1y * e2x
        ss = nx * nx + ny * ny + nz * nz
        ib = plsc.bitcast(ss, jnp.int32)
        ib = jnp.int32(0x5F3759DF) - (ib >> 1)
        yx = plsc.bitcast(ib, jnp.float32)
        h = ss * jnp.float32(0.5)
        yx = yx * (jnp.float32(1.5) - h * yx * yx)
        yx = yx * (jnp.float32(1.5) - h * yx * yx)
        yx = yx * (jnp.float32(1.5) - h * yx * yx)
        norm = ss * yx
        scale = jnp.float32(1.0) / (norm + jnp.float32(1e-12))
        nx = nx * scale
        ny = ny * scale
        nz = nz * scale
        nb = -(cx * nx + cy * ny + cz * nz)
        pn = cx * cx + cy * cy + cz * cz
        lanes = lax.iota(jnp.int32, 16)
        gidx = base + s + lanes
        pn = jnp.where(gidx < _NF, pn, jnp.float32(1e30))
        r0[pl.ds(s, 16)] = cx
        r1[pl.ds(s, 16)] = cy
        r2[pl.ds(s, 16)] = cz
        r3[pl.ds(s, 16)] = pn
        rowbase = (s + lanes) * _TW
        plsc.store_scatter(tbl_v, [rowbase], nx)
        plsc.store_scatter(tbl_v, [rowbase + 1], ny)
        plsc.store_scatter(tbl_v, [rowbase + 2], nz)
        plsc.store_scatter(tbl_v, [rowbase + 3], nb)
        return carry

    lax.fori_loop(0, _NSTEP, step, 0)
    pltpu.sync_copy(r0, o0.at[pl.ds(base, _FPT)])
    pltpu.sync_copy(r1, o1.at[pl.ds(base, _FPT)])
    pltpu.sync_copy(r2, o2.at[pl.ds(base, _FPT)])
    pltpu.sync_copy(r3, o3.at[pl.ds(base, _FPT)])
    pltpu.sync_copy(tbl_v, otbl.at[pl.ds(base * _TW, _FPT * _TW)])


_face_prep = pl.kernel(
    _face_prep_body,
    ([jax.ShapeDtypeStruct((_FPAD,), jnp.float32)] * 4
     + [jax.ShapeDtypeStruct((_FPAD * _TW,), jnp.float32)]),
    mesh=plsc.VectorSubcoreMesh(core_axis_name="c", subcore_axis_name="s"),
    compiler_params=pltpu.CompilerParams(needs_layout_passes=False),
    scratch_types=(
        [pltpu.VMEM((_NV * 3,), jnp.float32)]
        + [pltpu.VMEM((_FPT,), jnp.int32)] * 3
        + [pltpu.VMEM((_FPT,), jnp.float32)] * 4
        + [pltpu.VMEM((_FPT * _TW,), jnp.float32)]
    ),
)


def _knn_scan_body(q_ref, ct_ref, pn_ref, idx_ref, ms_ref, bs_ref, bi_ref):
    j = pl.program_id(1)
    q = q_ref[...]
    qx = q[:, 0:1]
    qy = q[:, 1:2]
    qz = q[:, 2:3]
    qn = qx * qx + qy * qy + qz * qz
    mm = lax.dot_general(q, ct_ref[...], (((1,), (0,)), ((), ())),
                         preferred_element_type=jnp.float32)
    s_blk = (qn + pn_ref[...]) - 2.0 * mm
    m = jnp.min(s_blk, axis=1, keepdims=True)
    lanei = lax.broadcasted_iota(jnp.int32, (_NQB, _FB), 1)
    lidx = jnp.min(jnp.where(s_blk == m, lanei, jnp.int32(0x7FFFFFFF)),
                   axis=1, keepdims=True)
    gidx = lidx + j * _FB

    @pl.when(j == 0)
    def _():
        bs_ref[...] = m
        bi_ref[...] = gidx

    @pl.when(j > 0)
    def _():
        upd = m < bs_ref[...]
        bs_ref[...] = jnp.where(upd, m, bs_ref[...])
        bi_ref[...] = jnp.where(upd, gidx, bi_ref[...])

    @pl.when(j == _NFB - 1)
    def _():
        idx_ref[...] = bi_ref[...]
        ms_ref[...] = bs_ref[...]


_knn_scan = pl.pallas_call(
    _knn_scan_body,
    grid=(_NQBLKS, _NFB),
    in_specs=[
        pl.BlockSpec((_NQB, 3), lambda i, j: (i, 0)),
        pl.BlockSpec((3, _FB), lambda i, j: (0, j)),
        pl.BlockSpec((1, _FB), lambda i, j: (0, j)),
    ],
    out_specs=[pl.BlockSpec((_NQB, 1), lambda i, j: (i, 0)),
               pl.BlockSpec((_NQB, 1), lambda i, j: (i, 0))],
    out_shape=[jax.ShapeDtypeStruct((_NQPAD, 1), jnp.int32),
               jax.ShapeDtypeStruct((_NQPAD, 1), jnp.float32)],
    scratch_shapes=[pltpu.VMEM((_NQB, 1), jnp.float32),
                    pltpu.VMEM((_NQB, 1), jnp.int32)],
)


def _gather_loss_body(tbl_hbm, idx_hbm, qx_hbm, qy_hbm, qz_hbm,
                      part_hbm,
                      idx_v, rows_v, qx_v, qy_v, qz_v, part_v, sem):
    wid = lax.axis_index("s") * 2 + lax.axis_index("c")
    base = wid * _QPT
    pltpu.sync_copy(idx_hbm.at[wid], idx_v)
    pltpu.sync_copy(qx_hbm.at[pl.ds(base, _QPT)], qx_v)
    pltpu.sync_copy(qy_hbm.at[pl.ds(base, _QPT)], qy_v)
    pltpu.sync_copy(qz_hbm.at[pl.ds(base, _QPT)], qz_v)
    copies = [
        pltpu.async_copy(tbl_hbm.at[idx_v.at[c]],
                         rows_v.at[pl.ds(c * _GW, _GW)], sem)
        for c in range(_GROWS)
    ]
    for cp in copies:
        cp.wait()

    def step(t, acc):
        acc_l, acc_c = acc
        s = t * 16
        lanes = lax.iota(jnp.int32, 16)
        row = s + lanes
        col0 = lanes * 0
        nx = plsc.load_gather(rows_v, [row, col0])
        ny = plsc.load_gather(rows_v, [row, col0 + 1])
        nz = plsc.load_gather(rows_v, [row, col0 + 2])
        nb = plsc.load_gather(rows_v, [row, col0 + 3])
        qx = qx_v[pl.ds(s, 16)]
        qy = qy_v[pl.ds(s, 16)]
        qz = qz_v[pl.ds(s, 16)]
        d = qx * nx + qy * ny + qz * nz + nb
        interp = jnp.maximum(jnp.float32(_EPS) - d, jnp.float32(0.0))
        valid = (base + s + lanes) < _NQ
        interp = jnp.where(valid, interp, jnp.float32(0.0))
        acc_l = acc_l + interp * interp * interp
        acc_c = acc_c + jnp.where(interp > jnp.float32(0.0),
                                  jnp.float32(1.0), jnp.float32(0.0))
        return (acc_l, acc_c)

    z = jnp.zeros((16,), jnp.float32)
    acc_l, acc_c = lax.fori_loop(0, _QSTEP, step, (z, z))
    part_v[pl.ds(0, 16)] = acc_l
    part_v[pl.ds(16, 16)] = acc_c
    pltpu.sync_copy(part_v, part_hbm.at[wid])


_gather_loss = pl.kernel(
    _gather_loss_body,
    jax.ShapeDtypeStruct((_NTILES, 32), jnp.float32),
    mesh=plsc.VectorSubcoreMesh(core_axis_name="c", subcore_axis_name="s"),
    compiler_params=pltpu.CompilerParams(needs_layout_passes=False,
                                         use_tc_tiling_on_sc=False),
    scratch_types=[
        pltpu.VMEM((_GROWS, _GW), jnp.int32),
        pltpu.VMEM((_QPT, _TW), jnp.float32),
        pltpu.VMEM((_QPT,), jnp.float32),
        pltpu.VMEM((_QPT,), jnp.float32),
        pltpu.VMEM((_QPT,), jnp.float32),
        pltpu.VMEM((32,), jnp.float32),
        pltpu.SemaphoreType.DMA,
    ],
)


def _combine_body(part_ref, loss_ref, perc_ref):
    p = part_ref[...]
    loss_ref[0, 0] = jnp.sum(p[:, 0:16])
    perc_ref[0, 0] = jnp.sum(p[:, 16:32]) * jnp.float32(1.0 / _NQ)


_combine = pl.pallas_call(
    _combine_body,
    out_specs=[pl.BlockSpec(memory_space=pltpu.SMEM),
               pl.BlockSpec(memory_space=pltpu.SMEM)],
    out_shape=[jax.ShapeDtypeStruct((1, 1), jnp.float32),
               jax.ShapeDtypeStruct((1, 1), jnp.float32)],
)


def kernel(obstacle_pos, pred_pos, obstacle_faces):
    pos_flat = obstacle_pos.reshape(-1)
    fpad = jnp.pad(obstacle_faces, ((0, _FPAD - _NF), (0, 0)))
    f0 = fpad[:, 0]
    f1 = fpad[:, 1]
    f2 = fpad[:, 2]
    c0, c1, c2, c3, tbl = _face_prep(pos_flat, f0, f1, f2)
    qpad = jnp.pad(pred_pos, ((0, _NQPAD - _NQ), (0, 0)))
    ct = jnp.concatenate([c0.reshape(1, _FPAD), c1.reshape(1, _FPAD),
                          c2.reshape(1, _FPAD)], axis=0)
    idx2d, _ = _knn_scan(qpad, ct, c3.reshape(1, _FPAD))
    parts = _gather_loss(tbl.reshape(_FPAD, _TW),
                         idx2d.reshape(_NTILES, _GROWS, _GW),
                         qpad[:, 0], qpad[:, 1], qpad[:, 2])
    loss2, perc2 = _combine(parts)
    return loss2[0, 0], perc2[0, 0]


# speedup vs baseline: 1.8322x; 1.3121x over previous
import jax
import jax.numpy as jnp
from jax import lax
from jax.experimental import pallas as pl
from jax.experimental.pallas import tpu as pltpu
from jax.experimental.pallas import tpu_sc as plsc

_EPS = 1e-3
_NV = 15000
_NF = 30000
_NQ = 50000

_NTILES = 32
_FPAD = 30720
_FPT = _FPAD // _NTILES
_NSTEP = _FPT // 16
_TW = 16

_NQB = 512
_NQPAD = 50176
_NQBLKS = _NQPAD // _NQB
_FB = 3072
_NFB = _FPAD // _FB

_QPT = _NQPAD // _NTILES
_GW = 112
_GROWS = _QPT // _GW
_QSTEP = _QPT // 16


def _face_prep_body(pos_hbm, f0_hbm, f1_hbm, f2_hbm,
                    o0, o1, o2, o3, otbl,
                    pos_v, f0_v, f1_v, f2_v,
                    r0, r1, r2, r3, tbl_v):
    wid = lax.axis_index("s") * 2 + lax.axis_index("c")
    base = wid * _FPT
    pltpu.sync_copy(pos_hbm, pos_v)
    pltpu.sync_copy(f0_hbm.at[pl.ds(base, _FPT)], f0_v)
    pltpu.sync_copy(f1_hbm.at[pl.ds(base, _FPT)], f1_v)
    pltpu.sync_copy(f2_hbm.at[pl.ds(base, _FPT)], f2_v)

    def step(i, carry):
        s = i * 16
        v0 = f0_v[pl.ds(s, 16)] * 3
        v1 = f1_v[pl.ds(s, 16)] * 3
        v2 = f2_v[pl.ds(s, 16)] * 3
        x0 = plsc.load_gather(pos_v, [v0])
        y0 = plsc.load_gather(pos_v, [v0 + 1])
        z0 = plsc.load_gather(pos_v, [v0 + 2])
        x1 = plsc.load_gather(pos_v, [v1])
        y1 = plsc.load_gather(pos_v, [v1 + 1])
        z1 = plsc.load_gather(pos_v, [v1 + 2])
        x2 = plsc.load_gather(pos_v, [v2])
        y2 = plsc.load_gather(pos_v, [v2 + 1])
        z2 = plsc.load_gather(pos_v, [v2 + 2])
        third = jnp.float32(1.0 / 3.0)
        cx = (x0 + x1 + x2) * third
        cy = (y0 + y1 + y2) * third
        cz = (z0 + z1 + z2) * third
        e1x = x1 - x0
        e1y = y1 - y0
        e1z = z1 - z0
        e2x = x2 - x0
        e2y = y2 - y0
        e2z = z2 - z0
        nx = e1y * e2z - e1z * e2y
        ny = e1z * e2x - e1x * e2z
        nz = e1x * e2y - e1y * e2x
        ss = nx * nx + ny * ny + nz * nz
        ib = plsc.bitcast(ss, jnp.int32)
        ib = jnp.int32(0x5F3759DF) - (ib >> 1)
        yx = plsc.bitcast(ib, jnp.float32)
        h = ss * jnp.float32(0.5)
        yx = yx * (jnp.float32(1.5) - h * yx * yx)
        yx = yx * (jnp.float32(1.5) - h * yx * yx)
        yx = yx * (jnp.float32(1.5) - h * yx * yx)
        norm = ss * yx
        scale = jnp.float32(1.0) / (norm + jnp.float32(1e-12))
        nx = nx * scale
        ny = ny * scale
        nz = nz * scale
        nb = -(cx * nx + cy * ny + cz * nz)
        pn = cx * cx + cy * cy + cz * cz
        lanes = lax.iota(jnp.int32, 16)
        gidx = base + s + lanes
        pn = jnp.where(gidx < _NF, pn, jnp.float32(1e30))
        r0[pl.ds(s, 16)] = cx
        r1[pl.ds(s, 16)] = cy
        r2[pl.ds(s, 16)] = cz
        r3[pl.ds(s, 16)] = pn
        rowbase = (s + lanes) * _TW
        plsc.store_scatter(tbl_v, [rowbase], nx)
        plsc.store_scatter(tbl_v, [rowbase + 1], ny)
        plsc.store_scatter(tbl_v, [rowbase + 2], nz)
        plsc.store_scatter(tbl_v, [rowbase + 3], nb)
        return carry

    lax.fori_loop(0, _NSTEP, step, 0)
    pltpu.sync_copy(r0, o0.at[pl.ds(base, _FPT)])
    pltpu.sync_copy(r1, o1.at[pl.ds(base, _FPT)])
    pltpu.sync_copy(r2, o2.at[pl.ds(base, _FPT)])
    pltpu.sync_copy(r3, o3.at[pl.ds(base, _FPT)])
    pltpu.sync_copy(tbl_v, otbl.at[pl.ds(base * _TW, _FPT * _TW)])


_face_prep = pl.kernel(
    _face_prep_body,
    ([jax.ShapeDtypeStruct((_FPAD,), jnp.float32)] * 4
     + [jax.ShapeDtypeStruct((_FPAD * _TW,), jnp.float32)]),
    mesh=plsc.VectorSubcoreMesh(core_axis_name="c", subcore_axis_name="s"),
    compiler_params=pltpu.CompilerParams(needs_layout_passes=False),
    scratch_types=(
        [pltpu.VMEM((_NV * 3,), jnp.float32)]
        + [pltpu.VMEM((_FPT,), jnp.int32)] * 3
        + [pltpu.VMEM((_FPT,), jnp.float32)] * 4
        + [pltpu.VMEM((_FPT * _TW,), jnp.float32)]
    ),
)


def _knn_scan_body(q_ref, ct_ref, pn_ref, idx_ref, ms_ref, bs_ref, bi_ref):
    j = pl.program_id(1)
    q = q_ref[...]
    qx = q[:, 0:1]
    qy = q[:, 1:2]
    qz = q[:, 2:3]
    qn = qx * qx + qy * qy + qz * qz
    mm = lax.dot_general(q, ct_ref[...], (((1,), (0,)), ((), ())),
                         preferred_element_type=jnp.float32)
    s_blk = (qn + pn_ref[...]) - 2.0 * mm
    m = jnp.min(s_blk, axis=1, keepdims=True)
    lanei = lax.broadcasted_iota(jnp.int32, (_NQB, _FB), 1)
    lidx = jnp.min(jnp.where(s_blk == m, lanei, jnp.int32(0x7FFFFFFF)),
                   axis=1, keepdims=True)
    gidx = lidx + j * _FB

    @pl.when(j == 0)
    def _():
        bs_ref[...] = m
        bi_ref[...] = gidx

    @pl.when(j > 0)
    def _():
        upd = m < bs_ref[...]
        bs_ref[...] = jnp.where(upd, m, bs_ref[...])
        bi_ref[...] = jnp.where(upd, gidx, bi_ref[...])

    @pl.when(j == _NFB - 1)
    def _():
        idx_ref[...] = bi_ref[...]
        ms_ref[...] = bs_ref[...]


_knn_scan = pl.pallas_call(
    _knn_scan_body,
    grid=(_NQBLKS, _NFB),
    in_specs=[
        pl.BlockSpec((_NQB, 3), lambda i, j: (i, 0)),
        pl.BlockSpec((3, _FB), lambda i, j: (0, j)),
        pl.BlockSpec((1, _FB), lambda i, j: (0, j)),
    ],
    out_specs=[pl.BlockSpec((_NQB, 1), lambda i, j: (i, 0)),
               pl.BlockSpec((_NQB, 1), lambda i, j: (i, 0))],
    out_shape=[jax.ShapeDtypeStruct((_NQPAD, 1), jnp.int32),
               jax.ShapeDtypeStruct((_NQPAD, 1), jnp.float32)],
    scratch_shapes=[pltpu.VMEM((_NQB, 1), jnp.float32),
                    pltpu.VMEM((_NQB, 1), jnp.int32)],
)


def _gather_loss_body(tbl_hbm, idx_hbm, qx_hbm, qy_hbm, qz_hbm,
                      part_hbm,
                      idx_v, rows_v, qx_v, qy_v, qz_v, part_v, sem):
    wid = lax.axis_index("s") * 2 + lax.axis_index("c")
    base = wid * _QPT
    pltpu.sync_copy(idx_hbm.at[wid], idx_v)
    pltpu.sync_copy(qx_hbm.at[pl.ds(base, _QPT)], qx_v)
    pltpu.sync_copy(qy_hbm.at[pl.ds(base, _QPT)], qy_v)
    pltpu.sync_copy(qz_hbm.at[pl.ds(base, _QPT)], qz_v)
    copies = [
        pltpu.async_copy(tbl_hbm.at[idx_v.at[c]],
                         rows_v.at[pl.ds(c * _GW, _GW)], sem)
        for c in range(_GROWS)
    ]
    for cp in copies:
        cp.wait()

    def step(t, acc):
        acc_l, acc_c = acc
        s = t * 16
        lanes = lax.iota(jnp.int32, 16)
        row = s + lanes
        col0 = lanes * 0
        nx = plsc.load_gather(rows_v, [row, col0])
        ny = plsc.load_gather(rows_v, [row, col0 + 1])
        nz = plsc.load_gather(rows_v, [row, col0 + 2])
        nb = plsc.load_gather(rows_v, [row, col0 + 3])
        qx = qx_v[pl.ds(s, 16)]
        qy = qy_v[pl.ds(s, 16)]
        qz = qz_v[pl.ds(s, 16)]
        d = qx * nx + qy * ny + qz * nz + nb
        interp = jnp.maximum(jnp.float32(_EPS) - d, jnp.float32(0.0))
        valid = (base + s + lanes) < _NQ
        interp = jnp.where(valid, interp, jnp.float32(0.0))
        acc_l = acc_l + interp * interp * interp
        acc_c = acc_c + jnp.where(interp > jnp.float32(0.0),
                                  jnp.float32(1.0), jnp.float32(0.0))
        return (acc_l, acc_c)

    z = jnp.zeros((16,), jnp.float32)
    acc_l, acc_c = lax.fori_loop(0, _QSTEP, step, (z, z))
    part_v[pl.ds(0, 16)] = acc_l
    part_v[pl.ds(16, 16)] = acc_c
    pltpu.sync_copy(part_v, part_hbm.at[wid])


_gather_loss = pl.kernel(
    _gather_loss_body,
    jax.ShapeDtypeStruct((_NTILES, 32), jnp.float32),
    mesh=plsc.VectorSubcoreMesh(core_axis_name="c", subcore_axis_name="s"),
    compiler_params=pltpu.CompilerParams(needs_layout_passes=False,
                                         use_tc_tiling_on_sc=False),
    scratch_types=[
        pltpu.VMEM((_GROWS, _GW), jnp.int32),
        pltpu.VMEM((_QPT, _TW), jnp.float32),
        pltpu.VMEM((_QPT,), jnp.float32),
        pltpu.VMEM((_QPT,), jnp.float32),
        pltpu.VMEM((_QPT,), jnp.float32),
        pltpu.VMEM((32,), jnp.float32),
        pltpu.SemaphoreType.DMA,
    ],
)


def _combine_body(part_ref, loss_ref, perc_ref):
    p = part_ref[...]
    loss_ref[0, 0] = jnp.sum(p[:, 0:16])
    perc_ref[0, 0] = jnp.sum(p[:, 16:32]) * jnp.float32(1.0 / _NQ)


_combine = pl.pallas_call(
    _combine_body,
    out_specs=[pl.BlockSpec(memory_space=pltpu.SMEM),
               pl.BlockSpec(memory_space=pltpu.SMEM)],
    out_shape=[jax.ShapeDtypeStruct((1, 1), jnp.float32),
               jax.ShapeDtypeStruct((1, 1), jnp.float32)],
)


def kernel(obstacle_pos, pred_pos, obstacle_faces):
    pos_flat = obstacle_pos.reshape(-1)
    fpad = jnp.pad(obstacle_faces, ((0, _FPAD - _NF), (0, 0)))
    f0 = fpad[:, 0]
    f1 = fpad[:, 1]
    f2 = fpad[:, 2]
    c0, c1, c2, c3, tbl = _face_prep(pos_flat, f0, f1, f2)
    qpad = jnp.pad(pred_pos, ((0, _NQPAD - _NQ), (0, 0)))
    ct = jnp.concatenate([c0.reshape(1, _FPAD), c1.reshape(1, _FPAD),
                          c2.reshape(1, _FPAD)], axis=0)
    idx2d, _ = _knn_scan(qpad, ct, c3.reshape(1, _FPAD))
    parts = _gather_loss(tbl.reshape(_FPAD, _TW),
                         idx2d.reshape(_NTILES, _GROWS, _GW),
                         qpad[:, 0], qpad[:, 1], qpad[:, 2])
    loss2, perc2 = _combine(parts)
    return loss2[0, 0], perc2[0, 0]
